# baseline (speedup 1.0000x reference)
import functools

import jax
import jax.numpy as jnp
from jax import lax
from jax.experimental import pallas as pl
from jax.experimental.pallas import tpu as pltpu
from jax.experimental.pallas import tpu_sc as plsc

_N = 100000
_E = 6400000
_ND = 4
_ED = 3
_D = 8
_NC = 2
_NS = 16
_NW = _NC * _NS
_EPW = _E // _NW
_C = 2000
_NCHUNK = _EPW // _C
_NPAD = 100096
_NPT = _NPAD // _NS


@functools.partial(
    pl.kernel,
    out_type=jax.ShapeDtypeStruct((_NC, _NPAD, _D), jnp.float32),
    mesh=plsc.VectorSubcoreMesh(core_axis_name="c", subcore_axis_name="s"),
    compiler_params=pltpu.CompilerParams(needs_layout_passes=False,
                                         use_tc_tiling_on_sc=False),
    scratch_types=[
        pltpu.VMEM_SHARED((_NPAD, _D), jnp.float32),
        pltpu.VMEM((4, 16), jnp.float32),
        pltpu.VMEM((_C,), jnp.float32),
        pltpu.VMEM((_C,), jnp.float32),
        pltpu.VMEM((_C,), jnp.float32),
        pltpu.VMEM((_C,), jnp.int32),
        pltpu.VMEM((_C,), jnp.int32),
        pltpu.VMEM((_C, _D), jnp.float32),
        pltpu.VMEM((_C, _D), jnp.float32),
        pltpu.SemaphoreType.DMA,
    ],
)
def _edge_sc(node_hbm, x0_hbm, x1_hbm, x2_hbm, src_hbm, dst_hbm, wb_hbm,
             zero_hbm, out_hbm, acc_sh, wb_s, x0_v, x1_v, x2_v,
             sidx_v, didx_v, orig_v, t_v, sem):
    c = lax.axis_index("c")
    s = lax.axis_index("s")
    wid = s * _NC + c

    pltpu.sync_copy(wb_hbm, wb_s)
    pltpu.sync_copy(zero_hbm.at[pl.ds(0, _C)], t_v)
    pltpu.sync_copy(zero_hbm.at[pl.ds(s * _NPT, _NPT)],
                    acc_sh.at[pl.ds(s * _NPT, _NPT)])
    plsc.subcore_barrier()

    lanes = lax.iota(jnp.int32, 16)
    cols = [jnp.full((16,), j, jnp.int32) for j in range(_ND)]
    wrow = [wb_s[d] for d in range(_ED + 1)]
    w = [[wrow[d][k] for k in range(16)] for d in range(_ED)]
    bsc = [wrow[_ED][k] for k in range(16)]

    def chunk_body(i, _):
        base = wid * _EPW + i * _C
        pltpu.sync_copy(src_hbm.at[pl.ds(base, _C)], sidx_v)
        pltpu.sync_copy(dst_hbm.at[pl.ds(base, _C)], didx_v)
        pltpu.sync_copy(x0_hbm.at[pl.ds(base, _C)], x0_v)
        pltpu.sync_copy(x1_hbm.at[pl.ds(base, _C)], x1_v)
        pltpu.sync_copy(x2_hbm.at[pl.ds(base, _C)], x2_v)

        pltpu.async_copy(node_hbm.at[sidx_v], orig_v, sem).wait()

        @plsc.parallel_loop(0, _C // 16, unroll=4)
        def cmp_body(g):
            rows = lanes + g * 16
            x = [xv[pl.ds(g * 16, 16)] for xv in (x0_v, x1_v, x2_v)]
            o = [plsc.load_gather(orig_v, [rows, cols[j]])
                 for j in range(_ND)]
            for ii in range(_ND):
                ti = None
                for j in range(_ND):
                    k = 4 * ii + j
                    a = x[0] * w[0][k] + x[1] * w[1][k] + x[2] * w[2][k] \
                        + bsc[k]
                    term = a * o[j]
                    ti = term if ti is None else ti + term
                plsc.store_scatter(t_v, [rows, cols[ii]], ti)

        pltpu.sync_copy(t_v, acc_sh.at[didx_v], add=True)
        return 0

    lax.fori_loop(0, _NCHUNK, chunk_body, 0, unroll=False)

    plsc.subcore_barrier()
    pltpu.sync_copy(acc_sh.at[pl.ds(s * _NPT, _NPT)],
                    out_hbm.at[c, pl.ds(s * _NPT, _NPT)])


def _combine_body(x_ref, o_ref):
    o_ref[...] = x_ref[0] + x_ref[1]


_combine = pl.pallas_call(
    _combine_body,
    out_shape=jax.ShapeDtypeStruct((_NPAD * _D // 128, 128), jnp.float32),
)


def kernel(node_features, edge_features, pair_indices, edge_kernel, bias):
    wb = jnp.concatenate([edge_kernel, bias[None, :]], axis=0)
    pair2 = pair_indices.astype(jnp.int32)
    src = pair2[:, 0]
    dst = pair2[:, 1]
    x0 = edge_features[:, 0]
    x1 = edge_features[:, 1]
    x2 = edge_features[:, 2]
    node_pad = jnp.pad(node_features, ((0, _NPAD - _N), (0, _D - _ND)))
    zeros = jnp.zeros((_NPAD, _D), jnp.float32)
    parts = _edge_sc(node_pad, x0, x1, x2, src, dst, wb, zeros)
    out = _combine(parts.reshape(_NC, _NPAD * _D // 128, 128))
    return out.reshape(_NPAD, _D)[:_N, :_ND]


# speedup vs baseline: 1.4348x; 1.4348x over previous
import functools

import jax
import jax.numpy as jnp
from jax import lax
from jax.experimental import pallas as pl
from jax.experimental.pallas import tpu as pltpu
from jax.experimental.pallas import tpu_sc as plsc

_N = 100000
_E = 6400000
_ND = 4
_ED = 3
_D = 8
_NC = 2
_NS = 16
_NW = _NC * _NS
_EPW = _E // _NW
_C = 2000
_NCHUNK = _EPW // _C
_NPAD = 100096
_NPT = _NPAD // _NS


@functools.partial(
    pl.kernel,
    out_type=jax.ShapeDtypeStruct((_NC, _NPAD, _D), jnp.float32),
    mesh=plsc.VectorSubcoreMesh(core_axis_name="c", subcore_axis_name="s"),
    compiler_params=pltpu.CompilerParams(needs_layout_passes=False,
                                         use_tc_tiling_on_sc=False),
    scratch_types=[
        pltpu.VMEM_SHARED((_NPAD, _D), jnp.float32),
        pltpu.VMEM((4, 16), jnp.float32),
        pltpu.VMEM((_C,), jnp.float32),
        pltpu.VMEM((_C,), jnp.float32),
        pltpu.VMEM((_C,), jnp.float32),
        pltpu.VMEM((_C,), jnp.int32),
        pltpu.VMEM((_C,), jnp.int32),
        pltpu.VMEM((_C, _D), jnp.float32),
        pltpu.VMEM((_C, _D), jnp.float32),
        pltpu.SemaphoreType.DMA,
        pltpu.SemaphoreType.DMA,
    ],
)
def _edge_sc(node_hbm, x0_hbm, x1_hbm, x2_hbm, src_hbm, dst_hbm, wb_hbm,
             zero_hbm, out_hbm, acc_sh, wb_s, x0_v, x1_v, x2_v,
             sidx_v, didx_v, orig_v, t_v, sem, lsem):
    c = lax.axis_index("c")
    s = lax.axis_index("s")
    wid = s * _NC + c

    pltpu.sync_copy(wb_hbm, wb_s)
    pltpu.sync_copy(zero_hbm.at[pl.ds(0, _C)], t_v)
    pltpu.sync_copy(zero_hbm.at[pl.ds(s * _NPT, _NPT)],
                    acc_sh.at[pl.ds(s * _NPT, _NPT)])
    plsc.subcore_barrier()

    lanes = lax.iota(jnp.int32, 16)
    cols = [jnp.full((16,), j, jnp.int32) for j in range(_ND)]
    wrow = [wb_s[d] for d in range(_ED + 1)]
    w = [[wrow[d][k] for k in range(16)] for d in range(_ED)]
    bsc = [wrow[_ED][k] for k in range(16)]

    def chunk_body(i, _):
        base = wid * _EPW + i * _C
        d_src = pltpu.async_copy(src_hbm.at[pl.ds(base, _C)], sidx_v, lsem)
        d_dst = pltpu.async_copy(dst_hbm.at[pl.ds(base, _C)], didx_v, lsem)
        d_x0 = pltpu.async_copy(x0_hbm.at[pl.ds(base, _C)], x0_v, lsem)
        d_x1 = pltpu.async_copy(x1_hbm.at[pl.ds(base, _C)], x1_v, lsem)
        d_x2 = pltpu.async_copy(x2_hbm.at[pl.ds(base, _C)], x2_v, lsem)
        d_src.wait()
        d_g = pltpu.async_copy(node_hbm.at[sidx_v], orig_v, sem)
        d_dst.wait()
        d_x0.wait()
        d_x1.wait()
        d_x2.wait()
        d_g.wait()

        def cmp_body(g, _):
            rows = lanes + g * 16
            x = [xv[pl.ds(g * 16, 16)] for xv in (x0_v, x1_v, x2_v)]
            o = [plsc.load_gather(orig_v, [rows, cols[j]])
                 for j in range(_ND)]
            for ii in range(_ND):
                ti = None
                for j in range(_ND):
                    k = 4 * ii + j
                    a = x[0] * w[0][k] + x[1] * w[1][k] + x[2] * w[2][k] \
                        + bsc[k]
                    term = a * o[j]
                    ti = term if ti is None else ti + term
                plsc.store_scatter(t_v, [rows, cols[ii]], ti)
            return 0

        lax.fori_loop(0, _C // 16, cmp_body, 0, unroll=4)

        pltpu.sync_copy(t_v, acc_sh.at[didx_v], add=True)
        return 0

    lax.fori_loop(0, _NCHUNK, chunk_body, 0, unroll=False)

    plsc.subcore_barrier()
    pltpu.sync_copy(acc_sh.at[pl.ds(s * _NPT, _NPT)],
                    out_hbm.at[c, pl.ds(s * _NPT, _NPT)])


def _combine_body(x_ref, o_ref):
    o_ref[...] = x_ref[0] + x_ref[1]


_combine = pl.pallas_call(
    _combine_body,
    out_shape=jax.ShapeDtypeStruct((_NPAD * _D // 128, 128), jnp.float32),
)


def kernel(node_features, edge_features, pair_indices, edge_kernel, bias):
    wb = jnp.concatenate([edge_kernel, bias[None, :]], axis=0)
    pair2 = pair_indices.astype(jnp.int32)
    src = pair2[:, 0]
    dst = pair2[:, 1]
    x0 = edge_features[:, 0]
    x1 = edge_features[:, 1]
    x2 = edge_features[:, 2]
    node_pad = jnp.pad(node_features, ((0, _NPAD - _N), (0, _D - _ND)))
    zeros = jnp.zeros((_NPAD, _D), jnp.float32)
    parts = _edge_sc(node_pad, x0, x1, x2, src, dst, wb, zeros)
    out = _combine(parts.reshape(_NC, _NPAD * _D // 128, 128))
    return out.reshape(_NPAD, _D)[:_N, :_ND]
